# Initial kernel scaffold; baseline (speedup 1.0000x reference)
import jax
import jax.numpy as jnp
from jax.experimental import pallas as pl


def kernel(query_up, key_up, head_idx):
    raise NotImplementedError("write your pallas kernel here")


# speedup vs baseline: 6.8103x; 6.8103x over previous
import functools

import numpy as np
import jax
import jax.numpy as jnp
from jax import lax
from jax.experimental import pallas as pl
from jax.experimental.pallas import tpu as pltpu
from jax.experimental.pallas import tpu_sc as plsc

B, L, D, K = 2, 2048, 64, 64
NS = 16
QPW = L // NS
LANES = 16
NCHUNK = L // LANES

ONES = np.int32(0x01010101)
HIGH = np.int32(-2139062144)


def _bit(d):
    w = 1 << (d & 31)
    return np.int32(w - (1 << 32) if w >= (1 << 31) else w)


def _pack_words(stage, jj):
    zero = jnp.zeros((LANES,), jnp.float32)
    zero_i = jnp.zeros((LANES,), jnp.int32)
    w0 = zero_i
    w1 = zero_i
    for d in range(32):
        v = stage[d, pl.ds(jj * LANES, LANES)]
        w0 = w0 | jnp.where(v > zero, jnp.full((LANES,), _bit(d)), zero_i)
    for d in range(32, 64):
        v = stage[d, pl.ds(jj * LANES, LANES)]
        w1 = w1 | jnp.where(v > zero, jnp.full((LANES,), _bit(d)), zero_i)
    return w0, w1


@functools.partial(
    pl.kernel,
    out_type=jax.ShapeDtypeStruct((B * L * K,), jnp.int32),
    mesh=plsc.VectorSubcoreMesh(core_axis_name="c", subcore_axis_name="s"),
    compiler_params=pltpu.CompilerParams(needs_layout_passes=False),
    scratch_types=[
        pltpu.VMEM((D, QPW), jnp.float32),
        pltpu.VMEM((L,), jnp.int32),
        pltpu.VMEM((L,), jnp.int32),
        pltpu.VMEM((QPW,), jnp.int32),
        pltpu.VMEM((QPW,), jnp.int32),
        pltpu.VMEM((QPW * K + LANES,), jnp.int32),
        pltpu.VMEM_SHARED((2 * L,), jnp.int32),
    ],
)
def _finder(qt_hbm, kt_hbm, out_hbm, stage, k0r, k1r, q0r, q1r, obuf, shared):
    c = lax.axis_index("c")
    s = lax.axis_index("s")
    base = s * QPW

    pltpu.sync_copy(kt_hbm.at[c, :, pl.ds(base, QPW)], stage)

    def pack_keys(jj, _):
        w0, w1 = _pack_words(stage, jj)
        k0r[pl.ds(base + jj * LANES, LANES)] = w0
        k1r[pl.ds(base + jj * LANES, LANES)] = w1
        return 0

    lax.fori_loop(0, QPW // LANES, pack_keys, 0)
    pltpu.sync_copy(k0r.at[pl.ds(base, QPW)], shared.at[pl.ds(base, QPW)])
    pltpu.sync_copy(k1r.at[pl.ds(base, QPW)], shared.at[pl.ds(L + base, QPW)])

    pltpu.sync_copy(qt_hbm.at[c, :, pl.ds(base, QPW)], stage)

    def pack_queries(jj, _):
        w0, w1 = _pack_words(stage, jj)
        q0r[pl.ds(jj * LANES, LANES)] = w0
        q1r[pl.ds(jj * LANES, LANES)] = w1
        return 0

    lax.fori_loop(0, QPW // LANES, pack_queries, 0)

    plsc.subcore_barrier()
    pltpu.sync_copy(shared.at[pl.ds(0, L)], k0r)
    pltpu.sync_copy(shared.at[pl.ds(L, L)], k1r)

    iota = lax.iota(jnp.int32, LANES)
    neg1 = jnp.full((LANES,), -1, jnp.int32)
    ones_v = jnp.full((LANES,), ONES, jnp.int32)
    high_v = jnp.full((LANES,), HIGH, jnp.int32)
    zero_v = jnp.zeros((LANES,), jnp.int32)

    def per_query_group(qg, _):
        qv0 = q0r[pl.ds(qg * LANES, LANES)]
        qv1 = q1r[pl.ds(qg * LANES, LANES)]
        for lane in range(LANES):
            q0v = lax.broadcast_in_dim(qv0[lane], (LANES,), ())
            q1v = lax.broadcast_in_dim(qv1[lane], (LANES,), ())
            row = (qg * LANES + lane) * K
            for r in range(K // LANES):
                obuf[pl.ds(row + r * LANES, LANES)] = neg1

            def chunk(cb, found, q0v=q0v, q1v=q1v, row=row):
                x0 = k0r[pl.ds(cb * LANES, LANES)] ^ q0v
                x1 = k1r[pl.ds(cb * LANES, LANES)] ^ q1v
                t = ((x0 - ones_v) & ~x0) | ((x1 - ones_v) & ~x1)
                m = (t & high_v) != zero_v
                cnt = plsc.all_reduce_population_count(m)[0]

                @pl.when(jnp.logical_and(found < K, cnt > 0))
                def _():
                    idxv = iota + lax.broadcast_in_dim(
                        cb * LANES, (LANES,), ())
                    plsc.store_compressed(
                        obuf.at[pl.ds(row + found, LANES)], idxv, mask=m)

                return found + cnt

            lax.fori_loop(0, NCHUNK, chunk, jnp.int32(0))
        return 0

    lax.fori_loop(0, QPW // LANES, per_query_group, 0)
    pltpu.sync_copy(obuf.at[pl.ds(0, QPW * K)],
                    out_hbm.at[pl.ds((c * L + base) * K, QPW * K)])


def kernel(query_up, key_up, head_idx):
    del head_idx
    qt = jnp.transpose(query_up, (0, 2, 1))
    kt = jnp.transpose(key_up, (0, 2, 1))
    return _finder(qt, kt).reshape(B, L, K)
